# baseline (speedup 1.0000x reference)
import functools

import jax
import jax.numpy as jnp
from jax import lax
from jax.experimental import pallas as pl
from jax.experimental.pallas import tpu as pltpu
from jax.experimental.pallas import tpu_sc as plsc

NC = 2
NS = 16
L = 16
NW = NC * NS

BATCH = 16384
DIM = 64
BPW = BATCH // NW
CHUNK = 128
NCHUNK = BPW // CHUNK
GPC = CHUNK // L


def _gmf_body(user_table, item_table, user, item, out,
              uidx_v, iidx_v, urid_v, irid_v, uoff_v, ioff_v,
              ubuf, ibuf, out_v, sem):
    wid = lax.axis_index("s") * NC + lax.axis_index("c")
    base = wid * BPW

    pltpu.sync_copy(user.at[pl.ds(base, BPW)], uidx_v)
    pltpu.sync_copy(item.at[pl.ds(base, BPW)], iidx_v)

    def prep(i, carry):
        s = pl.ds(i * L, L)
        u = uidx_v[s]
        v = iidx_v[s]
        urid_v[s] = u >> 1
        irid_v[s] = v >> 1
        uoff_v[s] = (u & 1) << 6
        ioff_v[s] = (v & 1) << 6
        return carry

    lax.fori_loop(0, BPW // L, prep, 0)

    def fire(j, b):
        cu = pltpu.async_copy(
            user_table.at[urid_v.at[pl.ds(j * CHUNK, CHUNK)]], ubuf.at[b], sem)
        ci = pltpu.async_copy(
            item_table.at[irid_v.at[pl.ds(j * CHUNK, CHUNK)]], ibuf.at[b], sem)
        return cu, ci

    def compute(j, b):
        bvec = jnp.full((L,), b, dtype=jnp.int32)
        for g in range(GPC):
            e = g * L + lax.iota(jnp.int32, L)
            uo = uoff_v[pl.ds(j * CHUNK + g * L, L)]
            io = ioff_v[pl.ds(j * CHUNK + g * L, L)]

            def dstep(d, acc):
                u = plsc.load_gather(ubuf, [bvec, e, uo + d])
                v = plsc.load_gather(ibuf, [bvec, e, io + d])
                return acc + u * v

            acc = lax.fori_loop(0, DIM, dstep, jnp.zeros((L,), jnp.float32))
            out_v[pl.ds(j * CHUNK + g * L, L)] = 1.0 / (1.0 + jnp.exp(-acc))

    pending = fire(0, 0)
    for j in range(NCHUNK):
        nxt = fire(j + 1, (j + 1) % 2) if j + 1 < NCHUNK else None
        for c in pending:
            c.wait()
        compute(j, j % 2)
        pending = nxt

    pltpu.sync_copy(out_v, out.at[pl.ds(base, BPW)])


_gmf = functools.partial(
    pl.kernel,
    out_type=jax.ShapeDtypeStruct((BATCH,), jnp.float32),
    mesh=plsc.VectorSubcoreMesh(core_axis_name="c", subcore_axis_name="s"),
    scratch_types=[
        pltpu.VMEM((BPW,), jnp.int32),
        pltpu.VMEM((BPW,), jnp.int32),
        pltpu.VMEM((BPW,), jnp.int32),
        pltpu.VMEM((BPW,), jnp.int32),
        pltpu.VMEM((BPW,), jnp.int32),
        pltpu.VMEM((BPW,), jnp.int32),
        pltpu.VMEM((2, CHUNK, 2 * DIM), jnp.float32),
        pltpu.VMEM((2, CHUNK, 2 * DIM), jnp.float32),
        pltpu.VMEM((BPW,), jnp.float32),
        pltpu.SemaphoreType.DMA,
    ],
    compiler_params=pltpu.CompilerParams(
        needs_layout_passes=False, use_tc_tiling_on_sc=True),
)(_gmf_body)


def kernel(user_table, item_table, user, item):
    ut = user_table.reshape(user_table.shape[0] // 2, 2 * DIM)
    it = item_table.reshape(item_table.shape[0] // 2, 2 * DIM)
    return _gmf(ut, it, user.astype(jnp.int32), item.astype(jnp.int32))
